# Initial kernel scaffold; baseline (speedup 1.0000x reference)
import jax
import jax.numpy as jnp
from jax.experimental import pallas as pl


def kernel(xyz_data, xyz_query, nn_idx):
    raise NotImplementedError("write your pallas kernel here")


# speedup vs baseline: 6.4558x; 6.4558x over previous
import functools
import math

import jax
import jax.numpy as jnp
from jax import lax
from jax.experimental import pallas as pl
from jax.experimental.pallas import tpu as pltpu
from jax.experimental.pallas import tpu_sc as plsc

NUM_CORES = 2
NUM_SUBCORES = 16
NUM_WORKERS = NUM_CORES * NUM_SUBCORES
LANES = 16
BLOCK = 2000

_C0 = math.sqrt(1.0 / (4.0 * math.pi))
_C1 = math.sqrt(3.0 / (4.0 * math.pi))
_C2M2 = math.sqrt(15.0 / (4.0 * math.pi))
_C20 = 0.25 * math.sqrt(5.0 / math.pi)
_C22 = 0.25 * math.sqrt(15.0 / math.pi)
_C3M3 = math.sqrt(35.0 / (32.0 * math.pi))
_C3M2 = 0.5 * math.sqrt(105.0 / math.pi)
_C3M1 = math.sqrt(21.0 / (32.0 * math.pi))
_C30 = 0.25 * math.sqrt(7.0 / math.pi)
_C32 = 0.25 * math.sqrt(105.0 / math.pi)


def _splat_f(v):
    return jnp.full((LANES,), v, jnp.float32)


def _splat_i(v):
    return jnp.full((LANES,), v, jnp.int32)


def _rsqrt_newton(s2):
    i = lax.bitcast_convert_type(s2, jnp.int32)
    seed = _splat_i(0x5F3759DF) - lax.shift_right_arithmetic(i, _splat_i(1))
    y = lax.bitcast_convert_type(seed, jnp.float32)
    half = _splat_f(0.5) * s2
    three_half = _splat_f(1.5)
    for _ in range(3):
        y = y * (three_half - half * y * y)
    return y


def _sh_coeffs(X, Y, Z):
    X2 = X * X
    Y2 = Y * Y
    Z2 = Z * Z
    XY = X * Y
    one = _splat_f(1.0)
    return [
        _splat_f(_C0),
        _splat_f(-_C1) * Y,
        _splat_f(_C1) * Z,
        _splat_f(-_C1) * X,
        _splat_f(_C2M2) * XY,
        _splat_f(-_C2M2) * (Y * Z),
        _splat_f(_C20) * (_splat_f(3.0) * Z2 - one),
        _splat_f(-_C2M2) * (X * Z),
        _splat_f(_C22) * (X2 - Y2),
        _splat_f(-_C3M3) * Y * (_splat_f(3.0) * X2 - Y2),
        _splat_f(_C3M2) * XY * Z,
        _splat_f(-_C3M1) * Y * (_splat_f(5.0) * Z2 - one),
        _splat_f(_C30) * Z * (_splat_f(5.0) * Z2 - _splat_f(3.0)),
        _splat_f(-_C3M1) * X * (_splat_f(5.0) * Z2 - one),
        _splat_f(_C32) * Z * (X2 - Y2),
        _splat_f(-_C3M3) * X * (X2 - _splat_f(3.0) * Y2),
    ]


def kernel(xyz_data, xyz_query, nn_idx):
    num_edges = nn_idx.shape[0]
    per_worker = num_edges // NUM_WORKERS
    assert per_worker * NUM_WORKERS == num_edges
    assert per_worker % BLOCK == 0
    nblocks = per_worker // BLOCK

    xd, yd, zd = (xyz_data[:, c] for c in range(3))
    xq, yq, zq = (xyz_query[:, c] for c in range(3))
    idx_q = nn_idx[:, 0]
    idx_d = nn_idx[:, 1]

    mesh = plsc.VectorSubcoreMesh(core_axis_name="c", subcore_axis_name="s")

    @functools.partial(
        pl.kernel,
        out_type=jax.ShapeDtypeStruct((num_edges * 16,), jnp.float32),
        mesh=mesh,
        scratch_types=[
            pltpu.VMEM((BLOCK,), jnp.int32),
            pltpu.VMEM((BLOCK,), jnp.int32),
            pltpu.VMEM((BLOCK,), jnp.float32),
            pltpu.VMEM((BLOCK,), jnp.float32),
            pltpu.VMEM((BLOCK,), jnp.float32),
            pltpu.VMEM((BLOCK,), jnp.float32),
            pltpu.VMEM((BLOCK,), jnp.float32),
            pltpu.VMEM((BLOCK,), jnp.float32),
            pltpu.VMEM((BLOCK * 16,), jnp.float32),
            pltpu.SemaphoreType.DMA,
        ],
        compiler_params=pltpu.CompilerParams(needs_layout_passes=False),
    )
    def sc_kernel(
        xd_hbm,
        yd_hbm,
        zd_hbm,
        xq_hbm,
        yq_hbm,
        zq_hbm,
        iq_hbm,
        id_hbm,
        out_hbm,
        iq_v,
        id_v,
        xd_v,
        yd_v,
        zd_v,
        xq_v,
        yq_v,
        zq_v,
        out_v,
        sem,
    ):
        wid = lax.axis_index("s") * NUM_CORES + lax.axis_index("c")
        lane = lax.iota(jnp.int32, 16)
        lane16 = lane * _splat_i(16)

        def block_body(b, _):
            base = wid * per_worker + b * BLOCK
            pltpu.sync_copy(iq_hbm.at[pl.ds(base, BLOCK)], iq_v)
            pltpu.sync_copy(id_hbm.at[pl.ds(base, BLOCK)], id_v)
            cps = [
                pltpu.async_copy(xd_hbm.at[id_v], xd_v, sem),
                pltpu.async_copy(yd_hbm.at[id_v], yd_v, sem),
                pltpu.async_copy(zd_hbm.at[id_v], zd_v, sem),
                pltpu.async_copy(xq_hbm.at[iq_v], xq_v, sem),
                pltpu.async_copy(yq_hbm.at[iq_v], yq_v, sem),
                pltpu.async_copy(zq_hbm.at[iq_v], zq_v, sem),
            ]
            for cp in cps:
                cp.wait()

            def vec_body(j, _):
                sl = pl.ds(j * LANES, LANES)
                dx = xd_v[sl] - xq_v[sl]
                dy = yd_v[sl] - yq_v[sl]
                dz = zd_v[sl] - zq_v[sl]
                s2 = dx * dx + dy * dy + dz * dz
                rinv = _rsqrt_newton(s2)
                coeffs = _sh_coeffs(dx * rinv, dy * rinv, dz * rinv)
                obase = jnp.full((LANES,), j * 256, jnp.int32) + lane16
                for c in range(16):
                    plsc.store_scatter(out_v, [obase + _splat_i(c)], coeffs[c])
                return 0

            lax.fori_loop(0, BLOCK // LANES, vec_body, 0)
            pltpu.sync_copy(out_v, out_hbm.at[pl.ds(base * 16, BLOCK * 16)])
            return 0

        lax.fori_loop(0, nblocks, block_body, 0)

    out = sc_kernel(xd, yd, zd, xq, yq, zq, idx_q, idx_d)
    return out.reshape(num_edges, 16)


# speedup vs baseline: 7.3194x; 1.1338x over previous
import functools
import math

import jax
import jax.numpy as jnp
from jax import lax
from jax.experimental import pallas as pl
from jax.experimental.pallas import tpu as pltpu
from jax.experimental.pallas import tpu_sc as plsc

NUM_CORES = 2
NUM_SUBCORES = 16
NUM_WORKERS = NUM_CORES * NUM_SUBCORES
LANES = 16
BLOCK = 2000

_C0 = math.sqrt(1.0 / (4.0 * math.pi))
_C1 = math.sqrt(3.0 / (4.0 * math.pi))
_C2M2 = math.sqrt(15.0 / (4.0 * math.pi))
_C20 = 0.25 * math.sqrt(5.0 / math.pi)
_C22 = 0.25 * math.sqrt(15.0 / math.pi)
_C3M3 = math.sqrt(35.0 / (32.0 * math.pi))
_C3M2 = 0.5 * math.sqrt(105.0 / math.pi)
_C3M1 = math.sqrt(21.0 / (32.0 * math.pi))
_C30 = 0.25 * math.sqrt(7.0 / math.pi)
_C32 = 0.25 * math.sqrt(105.0 / math.pi)


def _splat_f(v):
    return jnp.full((LANES,), v, jnp.float32)


def _splat_i(v):
    return jnp.full((LANES,), v, jnp.int32)


def _rsqrt_newton(s2):
    i = lax.bitcast_convert_type(s2, jnp.int32)
    seed = _splat_i(0x5F3759DF) - lax.shift_right_arithmetic(i, _splat_i(1))
    y = lax.bitcast_convert_type(seed, jnp.float32)
    half = _splat_f(0.5) * s2
    three_half = _splat_f(1.5)
    for _ in range(2):
        y = y * (three_half - half * y * y)
    return y


def _sh_coeffs(X, Y, Z):
    X2 = X * X
    Y2 = Y * Y
    Z2 = Z * Z
    XY = X * Y
    one = _splat_f(1.0)
    return [
        _splat_f(_C0),
        _splat_f(-_C1) * Y,
        _splat_f(_C1) * Z,
        _splat_f(-_C1) * X,
        _splat_f(_C2M2) * XY,
        _splat_f(-_C2M2) * (Y * Z),
        _splat_f(_C20) * (_splat_f(3.0) * Z2 - one),
        _splat_f(-_C2M2) * (X * Z),
        _splat_f(_C22) * (X2 - Y2),
        _splat_f(-_C3M3) * Y * (_splat_f(3.0) * X2 - Y2),
        _splat_f(_C3M2) * XY * Z,
        _splat_f(-_C3M1) * Y * (_splat_f(5.0) * Z2 - one),
        _splat_f(_C30) * Z * (_splat_f(5.0) * Z2 - _splat_f(3.0)),
        _splat_f(-_C3M1) * X * (_splat_f(5.0) * Z2 - one),
        _splat_f(_C32) * Z * (X2 - Y2),
        _splat_f(-_C3M3) * X * (X2 - _splat_f(3.0) * Y2),
    ]


def kernel(xyz_data, xyz_query, nn_idx):
    num_edges = nn_idx.shape[0]
    per_worker = num_edges // NUM_WORKERS
    assert per_worker * NUM_WORKERS == num_edges
    assert per_worker % BLOCK == 0
    nblocks = per_worker // BLOCK

    xd, yd, zd = (xyz_data[:, c] for c in range(3))
    xq, yq, zq = (xyz_query[:, c] for c in range(3))
    idx_q = nn_idx[:, 0]
    idx_d = nn_idx[:, 1]

    mesh = plsc.VectorSubcoreMesh(core_axis_name="c", subcore_axis_name="s")

    scratch = (
        [pltpu.VMEM((BLOCK,), jnp.int32)] * 4
        + [pltpu.VMEM((BLOCK,), jnp.float32)] * 12
        + [pltpu.VMEM((BLOCK * 16,), jnp.float32)] * 2
        + [pltpu.SemaphoreType.DMA] * 4
    )

    @functools.partial(
        pl.kernel,
        out_type=jax.ShapeDtypeStruct((num_edges * 16,), jnp.float32),
        mesh=mesh,
        scratch_types=scratch,
        compiler_params=pltpu.CompilerParams(needs_layout_passes=False),
    )
    def sc_kernel(
        xd_hbm, yd_hbm, zd_hbm, xq_hbm, yq_hbm, zq_hbm, iq_hbm, id_hbm, out_hbm,
        iq0, iq1, id0, id1,
        xd0, xd1, yd0, yd1, zd0, zd1, xq0, xq1, yq0, yq1, zq0, zq1,
        ov0, ov1,
        sg0, sg1, so0, so1,
    ):
        wid = lax.axis_index("s") * NUM_CORES + lax.axis_index("c")
        lane = lax.iota(jnp.int32, 16)
        lane16 = lane * _splat_i(16)
        iq_v = (iq0, iq1)
        id_v = (id0, id1)
        planes = ((xd0, xd1), (yd0, yd1), (zd0, zd1),
                  (xq0, xq1), (yq0, yq1), (zq0, zq1))
        out_v = (ov0, ov1)
        sem_g = (sg0, sg1)
        sem_o = (so0, so1)
        tables = (xd_hbm, yd_hbm, zd_hbm, xq_hbm, yq_hbm, zq_hbm)

        def gather_args(g, s):
            for t, tab in enumerate(tables):
                idx = id_v[s] if t < 3 else iq_v[s]
                yield tab.at[idx], planes[t][s], sem_g[s]

        def fetch(g, s):
            base = wid * per_worker + g * BLOCK
            pltpu.sync_copy(iq_hbm.at[pl.ds(base, BLOCK)], iq_v[s])
            pltpu.sync_copy(id_hbm.at[pl.ds(base, BLOCK)], id_v[s])
            for src, dst, sem in gather_args(g, s):
                pltpu.async_copy(src, dst, sem)

        def drain_gathers(g, s):
            for src, dst, sem in gather_args(g, s):
                pltpu.make_async_copy(src, dst, sem).wait()

        def out_slice(g):
            base = wid * per_worker + g * BLOCK
            return out_hbm.at[pl.ds(base * 16, BLOCK * 16)]

        UNROLL = 2

        def compute(g, s):
            xdv, ydv, zdv = planes[0][s], planes[1][s], planes[2][s]
            xqv, yqv, zqv = planes[3][s], planes[4][s], planes[5][s]
            ov = out_v[s]

            def group(jj):
                sl = pl.ds(jj * LANES, LANES)
                dx = xdv[sl] - xqv[sl]
                dy = ydv[sl] - yqv[sl]
                dz = zdv[sl] - zqv[sl]
                s2 = dx * dx + dy * dy + dz * dz
                rinv = _rsqrt_newton(s2)
                coeffs = _sh_coeffs(dx * rinv, dy * rinv, dz * rinv)
                obase = jnp.full((LANES,), jj * 256, jnp.int32) + lane16
                for c in range(16):
                    plsc.store_scatter(ov, [obase + _splat_i(c)], coeffs[c])

            def vec_body(j, _):
                for u in range(UNROLL):
                    group(j * UNROLL + u)
                return 0

            main_groups = (BLOCK // LANES) // UNROLL
            lax.fori_loop(0, main_groups, vec_body, 0)
            for jj in range(main_groups * UNROLL, BLOCK // LANES):
                group(jj)

        fetch(0, 0)
        for g in range(nblocks):
            s = g % 2
            if g + 1 < nblocks:
                fetch(g + 1, 1 - s)
            drain_gathers(g, s)
            if g >= 2:
                pltpu.make_async_copy(out_v[s], out_slice(g - 2), sem_o[s]).wait()
            compute(g, s)
            pltpu.async_copy(out_v[s], out_slice(g), sem_o[s])
        for g in (nblocks - 2, nblocks - 1):
            s = g % 2
            pltpu.make_async_copy(out_v[s], out_slice(g), sem_o[s]).wait()

    out = sc_kernel(xd, yd, zd, xq, yq, zq, idx_q, idx_d)
    return out.reshape(num_edges, 16)
